# baseline (speedup 1.0000x reference)
import functools

import jax
import jax.numpy as jnp
from jax import lax
from jax.experimental import pallas as pl
from jax.experimental.pallas import tpu as pltpu
from jax.experimental.pallas import tpu_sc as plsc

ROWS = 64
N = 32768
L = 16
NV = N // L
RADIX = 256
NCORES = 2
NSUB = 16
ROWS_PER_W = ROWS // (NCORES * NSUB)

_M31 = 0x7FFFFFFF


def _splat(val):
    return jnp.full((L,), val, dtype=jnp.int32)


def _keys(v):
    b = lax.bitcast_convert_type(v, jnp.int32)
    m = lax.shift_right_arithmetic(b, _splat(31))
    return b ^ (jnp.invert(m) & _splat(_M31))


def _unkeys(kd):
    m = lax.shift_right_arithmetic(kd, _splat(31))
    return lax.bitcast_convert_type(kd ^ (jnp.invert(m) & _splat(_M31)),
                                    jnp.float32)


def _load_i32(ref, i):
    v = ref[pl.ds(i * L, L)]
    if v.dtype == jnp.float32:
        v = lax.bitcast_convert_type(v, jnp.int32)
    return v


def _store_bits(ref, idx, kv):
    if ref.dtype == jnp.float32:
        kv = lax.bitcast_convert_type(kv, jnp.float32)
    plsc.store_scatter(ref, [idx], kv)


def _sc_body(z_hbm, out_hbm, zbuf, kbuf0, kbuf1, cnt, base):
    lanes = lax.iota(jnp.int32, 16)
    wid = lax.axis_index("s") * NCORES + lax.axis_index("c")
    ones = _splat(1)
    zeros = _splat(0)

    def radix_pass(in_ref, out_ref, shift, first, final):
        shift_v = _splat(shift)
        mask_v = _splat(RADIX - 1)

        def digits(i):
            kv = _load_i32(in_ref, i)
            if first:
                kv = _keys(lax.bitcast_convert_type(kv, jnp.float32))
            return kv, lax.shift_right_logical(kv, shift_v) & mask_v

        @plsc.parallel_loop(0, RADIX, unroll=8)
        def _zero(d):
            cnt[pl.ds(d * L, L)] = zeros

        @plsc.parallel_loop(0, NV, unroll=8)
        def _hist(i):
            _, d = digits(i)
            plsc.addupdate_scatter(cnt, [lax.shift_left(d, _splat(4)) | lanes],
                                   ones)

        @plsc.parallel_loop(0, RADIX, unroll=4, carry=jnp.int32(0))
        def _base(d, carry):
            sl = pl.ds(d * L, L)
            c = cnt[sl]
            incl = plsc.cumsum(c)
            base[sl] = incl - c + carry
            return carry + jnp.sum(c)

        def fetch(i):
            kv, d = digits(i)
            dl = lax.shift_left(d, _splat(4)) | lanes
            rk = plsc.load_gather(base, [dl])
            return (kv, dl, rk)

        def complete(slot):
            kv, dl, rk = slot
            plsc.store_scatter(base, [dl], rk + ones)
            if final:
                pos = rk
            else:
                pos = lax.shift_left(rk & _splat(NV - 1), _splat(4)) | \
                    lax.shift_right_logical(rk, _splat(11))
            _store_bits(out_ref, pos, kv)
            return dl

        def correct(slot, dl_done):
            kv, dl, rk = slot
            return (kv, dl, rk + jnp.where(dl == dl_done, 1, 0))

        s0 = fetch(0)
        s1 = fetch(1)
        s2 = fetch(2)

        def scat_body(i, carry):
            s0, s1, s2 = carry
            dl_done = complete(s0)
            s1 = correct(s1, dl_done)
            s2 = correct(s2, dl_done)
            s3 = fetch(i + 3)
            return (s1, s2, s3)
        s0, s1, s2 = lax.fori_loop(0, NV - 3, scat_body, (s0, s1, s2),
                                   unroll=4)
        dl_done = complete(s0)
        s1 = correct(s1, dl_done)
        s2 = correct(s2, dl_done)
        dl_done = complete(s1)
        s2 = correct(s2, dl_done)
        complete(s2)

    for r in range(ROWS_PER_W):
        row = wid * ROWS_PER_W + r
        pltpu.sync_copy(z_hbm.at[row], zbuf)

        radix_pass(zbuf, kbuf0, 0, True, False)
        radix_pass(kbuf0, kbuf1, 8, False, False)
        radix_pass(kbuf1, kbuf0, 16, False, False)
        radix_pass(kbuf0, kbuf1, 24, False, True)

        @plsc.parallel_loop(
            0, NV, unroll=4,
            carry=(jnp.float32(0.0), jnp.zeros((L,), jnp.float32)))
        def cs_carry(i, carry):
            csum, kacc = carry
            v = _unkeys(_load_i32(kbuf1, i))
            c = plsc.cumsum(v) + csum
            kbuf0[pl.ds(i * L, L)] = lax.bitcast_convert_type(c, jnp.int32)
            pos = (lanes + (i * L + 1)).astype(jnp.float32)
            pred = pos * v > c - 1.0
            return (csum + jnp.sum(v),
                    kacc + jnp.where(pred, 1.0, 0.0))
        _, kacc = cs_carry
        ksum = jnp.zeros((L,), jnp.float32) + jnp.sum(kacc)
        inv_k = jnp.ones((L,), jnp.float32) / ksum

        @plsc.parallel_loop(0, NV, unroll=8)
        def _out(i):
            sl = pl.ds(i * L, L)
            tau = (lax.bitcast_convert_type(kbuf0[sl], jnp.float32)
                   - 1.0) * inv_k
            kbuf1[sl] = jnp.maximum(zbuf[sl] - tau, 0.0)

        pltpu.sync_copy(kbuf1, out_hbm.at[row])


_sc_sparsemax = functools.partial(
    pl.kernel,
    out_type=jax.ShapeDtypeStruct((ROWS, N), jnp.float32),
    mesh=plsc.VectorSubcoreMesh(core_axis_name="c", subcore_axis_name="s"),
    compiler_params=pltpu.CompilerParams(needs_layout_passes=False),
    scratch_types=[
        pltpu.VMEM((N,), jnp.float32),
        pltpu.VMEM((N,), jnp.int32),
        pltpu.VMEM((N,), jnp.float32),
        pltpu.VMEM((RADIX * L,), jnp.int32),
        pltpu.VMEM((RADIX * L,), jnp.int32),
    ],
)(_sc_body)


def kernel(z):
    return _sc_sparsemax(z)


# speedup vs baseline: 1.0344x; 1.0344x over previous
import functools

import jax
import jax.numpy as jnp
from jax import lax
from jax.experimental import pallas as pl
from jax.experimental.pallas import tpu as pltpu
from jax.experimental.pallas import tpu_sc as plsc

ROWS = 64
N = 32768
L = 16
NV = N // L
RADIX = 256
NCORES = 2
NSUB = 16
ROWS_PER_W = ROWS // (NCORES * NSUB)

_M31 = 0x7FFFFFFF


def _splat(val):
    return jnp.full((L,), val, dtype=jnp.int32)


def _keys(v):
    b = lax.bitcast_convert_type(v, jnp.int32)
    m = lax.shift_right_arithmetic(b, _splat(31))
    return b ^ (jnp.invert(m) & _splat(_M31))


def _unkeys(kd):
    m = lax.shift_right_arithmetic(kd, _splat(31))
    return lax.bitcast_convert_type(kd ^ (jnp.invert(m) & _splat(_M31)),
                                    jnp.float32)


def _load_i32(ref, i):
    v = ref[pl.ds(i * L, L)]
    if v.dtype == jnp.float32:
        v = lax.bitcast_convert_type(v, jnp.int32)
    return v


def _store_bits(ref, idx, kv):
    if ref.dtype == jnp.float32:
        kv = lax.bitcast_convert_type(kv, jnp.float32)
    plsc.store_scatter(ref, [idx], kv)


def _sc_body(z_hbm, out_hbm, zbuf, kbuf0, kbuf1, cnt, base):
    lanes = lax.iota(jnp.int32, 16)
    wid = lax.axis_index("s") * NCORES + lax.axis_index("c")
    ones = _splat(1)
    zeros = _splat(0)

    def radix_pass(in_ref, out_ref, shift, first, final):

        def load_kv(i):
            kv = _load_i32(in_ref, i)
            if first:
                kv = _keys(lax.bitcast_convert_type(kv, jnp.float32))
            return kv

        def dladdr(kv):
            if shift == 0:
                d4 = lax.shift_left(kv & _splat(0xFF), _splat(4))
            else:
                d4 = lax.shift_right_logical(kv, _splat(shift - 4)) \
                    & _splat(0xFF0)
            return d4 | lanes

        @plsc.parallel_loop(0, NV, unroll=8)
        def _hist(i):
            plsc.addupdate_scatter(cnt, [dladdr(load_kv(i))], ones)

        @plsc.parallel_loop(0, RADIX, unroll=4, carry=jnp.int32(0))
        def _base(d, carry):
            sl = pl.ds(d * L, L)
            c = cnt[sl]
            incl = plsc.cumsum(c)
            base[sl] = incl - c + carry
            cnt[sl] = zeros
            return carry + jnp.sum(c)

        def fetch(i):
            kv = load_kv(i)
            dl = dladdr(kv)
            rk = plsc.load_gather(base, [dl])
            return (kv, dl, rk)

        def complete(slot):
            kv, dl, rk = slot
            plsc.store_scatter(base, [dl], rk + ones)
            if final:
                pos = rk
            else:
                pos = lax.shift_left(rk & _splat(NV - 1), _splat(4)) | \
                    lax.shift_right_logical(rk, _splat(11))
            _store_bits(out_ref, pos, kv)
            return dl

        def correct(slot, dl_done):
            kv, dl, rk = slot
            return (kv, dl, rk + jnp.where(dl == dl_done, 1, 0))

        s0 = fetch(0)
        s1 = fetch(1)
        s2 = fetch(2)

        def scat_body(i, carry):
            s0, s1, s2 = carry
            dl_done = complete(s0)
            s1 = correct(s1, dl_done)
            s2 = correct(s2, dl_done)
            s3 = fetch(i + 3)
            return (s1, s2, s3)
        s0, s1, s2 = lax.fori_loop(0, NV - 3, scat_body, (s0, s1, s2),
                                   unroll=8)
        dl_done = complete(s0)
        s1 = correct(s1, dl_done)
        s2 = correct(s2, dl_done)
        dl_done = complete(s1)
        s2 = correct(s2, dl_done)
        complete(s2)

    @plsc.parallel_loop(0, RADIX, unroll=8)
    def _zero_cnt(d):
        cnt[pl.ds(d * L, L)] = zeros

    for r in range(ROWS_PER_W):
        row = wid * ROWS_PER_W + r
        pltpu.sync_copy(z_hbm.at[row], zbuf)

        radix_pass(zbuf, kbuf0, 0, True, False)
        radix_pass(kbuf0, kbuf1, 8, False, False)
        radix_pass(kbuf1, kbuf0, 16, False, False)
        radix_pass(kbuf0, kbuf1, 24, False, True)

        @plsc.parallel_loop(
            0, NV, unroll=4,
            carry=(jnp.float32(0.0), jnp.zeros((L,), jnp.float32)))
        def cs_carry(i, carry):
            csum, kacc = carry
            v = _unkeys(_load_i32(kbuf1, i))
            c = plsc.cumsum(v) + csum
            kbuf0[pl.ds(i * L, L)] = lax.bitcast_convert_type(c, jnp.int32)
            pos = (lanes + (i * L + 1)).astype(jnp.float32)
            pred = pos * v > c - 1.0
            return (csum + jnp.sum(v),
                    kacc + jnp.where(pred, 1.0, 0.0))
        _, kacc = cs_carry
        ksum = jnp.zeros((L,), jnp.float32) + jnp.sum(kacc)
        inv_k = jnp.ones((L,), jnp.float32) / ksum

        @plsc.parallel_loop(0, NV, unroll=8)
        def _out(i):
            sl = pl.ds(i * L, L)
            tau = (lax.bitcast_convert_type(kbuf0[sl], jnp.float32)
                   - 1.0) * inv_k
            kbuf1[sl] = jnp.maximum(zbuf[sl] - tau, 0.0)

        pltpu.sync_copy(kbuf1, out_hbm.at[row])


_sc_sparsemax = functools.partial(
    pl.kernel,
    out_type=jax.ShapeDtypeStruct((ROWS, N), jnp.float32),
    mesh=plsc.VectorSubcoreMesh(core_axis_name="c", subcore_axis_name="s"),
    compiler_params=pltpu.CompilerParams(needs_layout_passes=False),
    scratch_types=[
        pltpu.VMEM((N,), jnp.float32),
        pltpu.VMEM((N,), jnp.int32),
        pltpu.VMEM((N,), jnp.float32),
        pltpu.VMEM((RADIX * L,), jnp.int32),
        pltpu.VMEM((RADIX * L,), jnp.int32),
    ],
)(_sc_body)


def kernel(z):
    return _sc_sparsemax(z)


# speedup vs baseline: 1.0474x; 1.0126x over previous
import functools

import jax
import jax.numpy as jnp
from jax import lax
from jax.experimental import pallas as pl
from jax.experimental.pallas import tpu as pltpu
from jax.experimental.pallas import tpu_sc as plsc

ROWS = 64
N = 32768
L = 16
NV = N // L
RADIX = 256
NCORES = 2
NSUB = 16
ROWS_PER_W = ROWS // (NCORES * NSUB)

_M31 = 0x7FFFFFFF


def _splat(val):
    return jnp.full((L,), val, dtype=jnp.int32)


def _keys(v):
    b = lax.bitcast_convert_type(v, jnp.int32)
    m = lax.shift_right_arithmetic(b, _splat(31))
    return b ^ (jnp.invert(m) & _splat(_M31))


def _unkeys(kd):
    m = lax.shift_right_arithmetic(kd, _splat(31))
    return lax.bitcast_convert_type(kd ^ (jnp.invert(m) & _splat(_M31)),
                                    jnp.float32)


def _load_i32(ref, i):
    v = ref[pl.ds(i * L, L)]
    if v.dtype == jnp.float32:
        v = lax.bitcast_convert_type(v, jnp.int32)
    return v


def _store_bits(ref, idx, kv):
    if ref.dtype == jnp.float32:
        kv = lax.bitcast_convert_type(kv, jnp.float32)
    plsc.store_scatter(ref, [idx], kv)


def _sc_body(z_hbm, out_hbm, zbuf, kbuf0, kbuf1, cnt, base):
    lanes = lax.iota(jnp.int32, 16)
    wid = lax.axis_index("s") * NCORES + lax.axis_index("c")
    ones = _splat(1)
    zeros = _splat(0)

    def radix_pass(in_ref, out_ref, shift, first, final):

        def load_kv(i):
            kv = _load_i32(in_ref, i)
            if first:
                kv = _keys(lax.bitcast_convert_type(kv, jnp.float32))
            return kv

        def dladdr(kv):
            if shift == 0:
                d4 = lax.shift_left(kv & _splat(0xFF), _splat(4))
            else:
                d4 = lax.shift_right_logical(kv, _splat(shift - 4)) \
                    & _splat(0xFF0)
            return d4 | lanes

        @plsc.parallel_loop(0, NV, unroll=8)
        def _hist(i):
            plsc.addupdate_scatter(cnt, [dladdr(load_kv(i))], ones)

        @plsc.parallel_loop(0, RADIX, unroll=4, carry=jnp.int32(0))
        def _base(d, carry):
            sl = pl.ds(d * L, L)
            c = cnt[sl]
            incl = plsc.cumsum(c)
            base[sl] = incl - c + carry
            cnt[sl] = zeros
            return carry + jnp.sum(c)

        def fetch(i):
            kv = load_kv(i)
            dl = dladdr(kv)
            rk = plsc.load_gather(base, [dl])
            return (kv, dl, rk)

        def complete(slot):
            kv, dl, rk = slot
            plsc.store_scatter(base, [dl], rk + ones)
            if final:
                pos = rk
            else:
                pos = lax.shift_left(rk & _splat(NV - 1), _splat(4)) | \
                    lax.shift_right_logical(rk, _splat(11))
            _store_bits(out_ref, pos, kv)
            return dl

        def correct(slot, dl_done):
            kv, dl, rk = slot
            return (kv, dl, rk + jnp.where(dl == dl_done, 1, 0))

        s0 = fetch(0)
        s1 = fetch(1)

        def scat_body(i, carry):
            s0, s1 = carry
            dl_done = complete(s0)
            s1 = correct(s1, dl_done)
            s2 = fetch(i + 2)
            return (s1, s2)
        s0, s1 = lax.fori_loop(0, NV - 2, scat_body, (s0, s1), unroll=8)
        dl_done = complete(s0)
        s1 = correct(s1, dl_done)
        complete(s1)

    @plsc.parallel_loop(0, RADIX, unroll=8)
    def _zero_cnt(d):
        cnt[pl.ds(d * L, L)] = zeros

    for r in range(ROWS_PER_W):
        row = wid * ROWS_PER_W + r
        pltpu.sync_copy(z_hbm.at[row], zbuf)

        radix_pass(zbuf, kbuf0, 0, True, False)
        radix_pass(kbuf0, kbuf1, 8, False, False)
        radix_pass(kbuf1, kbuf0, 16, False, False)
        radix_pass(kbuf0, kbuf1, 24, False, True)

        @plsc.parallel_loop(
            0, NV, unroll=4,
            carry=(jnp.float32(0.0), jnp.zeros((L,), jnp.float32)))
        def cs_carry(i, carry):
            csum, kacc = carry
            v = _unkeys(_load_i32(kbuf1, i))
            c = plsc.cumsum(v) + csum
            kbuf0[pl.ds(i * L, L)] = lax.bitcast_convert_type(c, jnp.int32)
            pos = (lanes + (i * L + 1)).astype(jnp.float32)
            pred = pos * v > c - 1.0
            return (csum + jnp.sum(v),
                    kacc + jnp.where(pred, 1.0, 0.0))
        _, kacc = cs_carry
        ksum = jnp.zeros((L,), jnp.float32) + jnp.sum(kacc)
        inv_k = jnp.ones((L,), jnp.float32) / ksum

        @plsc.parallel_loop(0, NV, unroll=8)
        def _out(i):
            sl = pl.ds(i * L, L)
            tau = (lax.bitcast_convert_type(kbuf0[sl], jnp.float32)
                   - 1.0) * inv_k
            kbuf1[sl] = jnp.maximum(zbuf[sl] - tau, 0.0)

        pltpu.sync_copy(kbuf1, out_hbm.at[row])


_sc_sparsemax = functools.partial(
    pl.kernel,
    out_type=jax.ShapeDtypeStruct((ROWS, N), jnp.float32),
    mesh=plsc.VectorSubcoreMesh(core_axis_name="c", subcore_axis_name="s"),
    compiler_params=pltpu.CompilerParams(needs_layout_passes=False),
    scratch_types=[
        pltpu.VMEM((N,), jnp.float32),
        pltpu.VMEM((N,), jnp.int32),
        pltpu.VMEM((N,), jnp.float32),
        pltpu.VMEM((RADIX * L,), jnp.int32),
        pltpu.VMEM((RADIX * L,), jnp.int32),
    ],
)(_sc_body)


def kernel(z):
    return _sc_sparsemax(z)
